# baseline (speedup 1.0000x reference)
import functools
import math

import jax
import jax.numpy as jnp
from jax import lax
from jax.experimental import pallas as pl
from jax.experimental.pallas import tpu as pltpu
from jax.experimental.pallas import tpu_sc as plsc

_N = 16384
_SIZE = 1000
_SMOOTH = 0.1
_CONF = 1.0 - _SMOOTH
_S = _SMOOTH / (_SIZE - 2)
_K = (_SIZE - 2) * _S * math.log(_S) + _CONF * math.log(_CONF)

_INFO = plsc.get_sparse_core_info()
_NC, _NS, _L = _INFO.num_cores, _INFO.num_subcores, _INFO.num_lanes
_NW = _NC * _NS

_C_TC = 808
_M = _SIZE - _C_TC
_SPB = 1024
_GRID = _N // _SPB

_SCHUNK = 4096
_CHUNKS = (_M // 8) * (_N // _SCHUNK)
_MAXK = -(-_CHUNKS // _NW)


def _tc_body(y_ref, tgt_ref, out_ref):
    i = pl.program_id(0)

    @pl.when(i == 0)
    def _init():
        out_ref[...] = jnp.zeros((1, 1), jnp.float32)

    y = y_ref[...]
    tgt = tgt_ref[...]
    valid = (tgt != 0)
    colsum = jnp.sum(y, axis=0, keepdims=True)
    x0 = y[0:1, :]
    classes = jax.lax.broadcasted_iota(jnp.int32, y.shape, 0)
    pick = jnp.sum(jnp.where(classes == tgt, y, 0.0), axis=0, keepdims=True)
    per_col = _K - _S * colsum + _S * x0 - (_CONF - _S) * pick
    out_ref[...] += jnp.sum(jnp.where(valid, per_col, 0.0)).reshape(1, 1)


@functools.partial(
    pl.kernel,
    mesh=plsc.VectorSubcoreMesh(core_axis_name="c", subcore_axis_name="s"),
    out_type=jax.ShapeDtypeStruct((_NW * _L,), jnp.float32),
    scratch_types=[
        pltpu.VMEM((_N,), jnp.int32),
        pltpu.VMEM((8, _SCHUNK), jnp.float32),
        pltpu.VMEM((8, _SCHUNK), jnp.float32),
        pltpu.VMEM((_L,), jnp.float32),
        pltpu.VMEM((_L,), jnp.float32),
        pltpu.VMEM((_L,), jnp.float32),
        pltpu.SemaphoreType.DMA,
        pltpu.SemaphoreType.DMA,
    ],
    compiler_params=pltpu.CompilerParams(
        use_tc_tiling_on_sc=True, needs_layout_passes=False),
)
def _sc_loss(y_hbm, tgt_hbm, out_hbm, tgt_v, buf0, buf1,
             acc_sum_r, acc_pick_r, res_v, sem0, sem1):
    wid = lax.axis_index("s") * _NC + lax.axis_index("c")
    pltpu.sync_copy(tgt_hbm.at[pl.ds(0, _N)], tgt_v)
    iota16 = lax.iota(jnp.int32, _L)
    zf = jnp.zeros((_L,), jnp.float32)
    acc_sum_r[...] = zf
    acc_pick_r[...] = zf

    def chunk_coords(cid):
        cb = cid // (_N // _SCHUNK)
        sq = lax.rem(cid, _N // _SCHUNK)
        return _C_TC + cb * 8, sq * _SCHUNK

    def start_dma(cid, buf, sem):
        c0, s0 = chunk_coords(cid)
        pltpu.async_copy(y_hbm.at[pl.ds(c0, 8), pl.ds(s0, _SCHUNK)], buf, sem)

    def compute_chunk(cid, buf, sem):
        pltpu.make_async_copy(
            y_hbm.at[pl.ds(0, 8), pl.ds(0, _SCHUNK)], buf, sem).wait()
        c0, s0 = chunk_coords(cid)

        def j_body(j, accs):
            a_sum, a_pick = accs
            sbase = j * _L
            tgt16 = tgt_v[pl.ds(s0 + sbase, _L)]
            valid = tgt16 != 0
            for r in range(8):
                v = buf[r, pl.ds(sbase, _L)]
                a_sum = a_sum + jnp.where(valid, v, zf)
            rel = tgt16 - c0
            inr = jnp.logical_and(rel >= 0, rel < 8)
            relc = jnp.where(inr, rel, jnp.zeros((_L,), jnp.int32))
            pv = plsc.load_gather(buf, [relc, sbase + iota16])
            a_pick = a_pick + jnp.where(inr, pv, zf)
            return (a_sum, a_pick)

        a_sum, a_pick = lax.fori_loop(
            0, _SCHUNK // _L, j_body, (acc_sum_r[...], acc_pick_r[...]))
        acc_sum_r[...] = a_sum
        acc_pick_r[...] = a_pick

    @pl.when(wid < _CHUNKS)
    def _prime():
        start_dma(wid, buf0, sem0)

    def k_body(k, carry):
        cid = wid + k * _NW
        nxt = cid + _NW
        parity = lax.rem(k, 2)

        @pl.when(jnp.logical_and(nxt < _CHUNKS, lax.rem(k + 1, 2) == 1))
        def _s1():
            start_dma(nxt, buf1, sem1)

        @pl.when(jnp.logical_and(nxt < _CHUNKS, lax.rem(k + 1, 2) == 0))
        def _s0():
            start_dma(nxt, buf0, sem0)

        @pl.when(jnp.logical_and(cid < _CHUNKS, parity == 0))
        def _c0():
            compute_chunk(cid, buf0, sem0)

        @pl.when(jnp.logical_and(cid < _CHUNKS, parity == 1))
        def _c1():
            compute_chunk(cid, buf1, sem1)

        return carry

    lax.fori_loop(0, _MAXK, k_body, 0)

    res_v[...] = -_S * acc_sum_r[...] - (_CONF - _S) * acc_pick_r[...]
    pltpu.sync_copy(res_v, out_hbm.at[pl.ds(wid * _L, _L)])


def kernel(x, target):
    y = x.T
    tgt32 = target.astype(jnp.int32)
    sc_parts = _sc_loss(y, tgt32)
    out = pl.pallas_call(
        _tc_body,
        grid=(_GRID,),
        in_specs=[
            pl.BlockSpec((_C_TC, _SPB), lambda i: (0, i)),
            pl.BlockSpec((1, _SPB), lambda i: (0, i)),
        ],
        out_specs=pl.BlockSpec((1, 1), lambda i: (0, 0)),
        out_shape=jax.ShapeDtypeStruct((1, 1), jnp.float32),
        compiler_params=pltpu.CompilerParams(
            dimension_semantics=("arbitrary",),
        ),
    )(y, tgt32.reshape(1, _N))
    return out[0, 0] + jnp.sum(sc_parts)


# speedup vs baseline: 2.0038x; 2.0038x over previous
import math

import jax
import jax.numpy as jnp
from jax.experimental import pallas as pl
from jax.experimental.pallas import tpu as pltpu

_N = 16384
_SIZE = 1000
_SMOOTH = 0.1
_CONF = 1.0 - _SMOOTH
_S = _SMOOTH / (_SIZE - 2)
_K = (_SIZE - 2) * _S * math.log(_S) + _CONF * math.log(_CONF)

_SAMPLES_PER_BLOCK = 2048
_GRID = _N // _SAMPLES_PER_BLOCK


def _tc_body(y_ref, tgt_ref, out_ref):
    i = pl.program_id(0)

    @pl.when(i == 0)
    def _init():
        out_ref[...] = jnp.zeros((1, 1), jnp.float32)

    y = y_ref[...]
    tgt = tgt_ref[...]
    valid = (tgt != 0)
    colsum = jnp.sum(y, axis=0, keepdims=True)
    x0 = y[0:1, :]
    classes = jax.lax.broadcasted_iota(jnp.int32, y.shape, 0)
    pick = jnp.sum(jnp.where(classes == tgt, y, 0.0), axis=0, keepdims=True)
    per_col = _K - _S * colsum + _S * x0 - (_CONF - _S) * pick
    out_ref[...] += jnp.sum(jnp.where(valid, per_col, 0.0)).reshape(1, 1)


def kernel(x, target):
    y = x.T
    tgt = target.astype(jnp.int32).reshape(1, _N)
    out = pl.pallas_call(
        _tc_body,
        grid=(_GRID,),
        in_specs=[
            pl.BlockSpec((_SIZE, _SAMPLES_PER_BLOCK), lambda i: (0, i)),
            pl.BlockSpec((1, _SAMPLES_PER_BLOCK), lambda i: (0, i)),
        ],
        out_specs=pl.BlockSpec((1, 1), lambda i: (0, 0)),
        out_shape=jax.ShapeDtypeStruct((1, 1), jnp.float32),
        compiler_params=pltpu.CompilerParams(
            dimension_semantics=("arbitrary",),
        ),
    )(y, tgt)
    return out[0, 0]
